# baseline (speedup 1.0000x reference)
import functools

import jax
import jax.numpy as jnp
from jax import lax
from jax.experimental import pallas as pl
from jax.experimental.pallas import tpu as pltpu
from jax.experimental.pallas import tpu_sc as plsc

ROWS, COLS = 4096, 4096
NUM_CODEBOOKS = 2
CODEBOOK_SIZE = 256
CENTROID_LEN = 8
RANK = 256

NC, NS, LANES = 2, 16, 16
NW = NC * NS
HROWS = ROWS // NUM_CODEBOOKS
N_CODES_HALF = HROWS * COLS // CENTROID_LEN
CODES_PER_W = N_CODES_HALF // NW

CODES_PER_CHUNK = 2048
CHUNK_ROWS = CODES_PER_CHUNK * CENTROID_LEN // COLS
NCHUNK = CODES_PER_W // CODES_PER_CHUNK
GROUPS_PER_CHUNK = CODES_PER_CHUNK // LANES
GROUPS_PER_ROW = COLS // (LANES * CENTROID_LEN)
ROWS_PER_W = HROWS // NW
TABLE_LEN = NUM_CODEBOOKS * CODEBOOK_SIZE * CENTROID_LEN


def _make_dequant_half(half):
    half_off = half * CODEBOOK_SIZE * CENTROID_LEN

    def body(cb_hbm, codes_hbm, dq_hbm,
             cb_v, codes_v0, codes_v1, out_v0, out_v1,
             sem_cb, sem_in0, sem_in1, sem_out0, sem_out1):
        wid = lax.axis_index("s") * NC + lax.axis_index("c")
        code_base = wid * CODES_PER_W
        row_base = wid * ROWS_PER_W

        pltpu.async_copy(cb_hbm, cb_v, sem_cb).wait()
        off8 = lax.iota(jnp.int32, LANES) * CENTROID_LEN

        sem_in = [sem_in0, sem_in1]
        sem_out = [sem_out0, sem_out1]
        codes_v = [codes_v0, codes_v1]
        out_v = [out_v0, out_v1]

        def start_in(ci):
            return pltpu.async_copy(
                codes_hbm.at[half,
                             pl.ds(code_base + ci * CODES_PER_CHUNK,
                                   CODES_PER_CHUNK)],
                codes_v[ci % 2], sem_in[ci % 2])

        def start_out(ci):
            return pltpu.async_copy(
                out_v[ci % 2],
                dq_hbm.at[pl.ds(row_base + ci * CHUNK_ROWS, CHUNK_ROWS)],
                sem_out[ci % 2])

        in_d = {0: start_in(0), 1: start_in(1)}
        out_d = {}
        for ci in range(NCHUNK):
            b = ci % 2
            in_d[ci].wait()
            if ci - 2 in out_d:
                out_d[ci - 2].wait()
            codes_b = codes_v[b]
            out_b = out_v[b]

            @plsc.parallel_loop(0, GROUPS_PER_CHUNK, unroll=4)
            def group(gi):
                codes16 = codes_b[pl.ds(gi * LANES, LANES)]
                eidx = codes16 * CENTROID_LEN + half_off
                row16 = jnp.full((LANES,), gi >> 5, jnp.int32)
                cbase = (gi & (GROUPS_PER_ROW - 1)) * (LANES * CENTROID_LEN)
                for k in range(CENTROID_LEN):
                    vals = plsc.load_gather(cb_v, [eidx + k])
                    plsc.store_scatter(
                        out_b, [row16, off8 + (cbase + k)], vals)

            out_d[ci] = start_out(ci)
            if ci + 2 < NCHUNK:
                in_d[ci + 2] = start_in(ci + 2)
        out_d[NCHUNK - 2].wait()
        out_d[NCHUNK - 1].wait()

    return functools.partial(
        pl.kernel,
        out_type=jax.ShapeDtypeStruct((HROWS, COLS), jnp.float32),
        mesh=plsc.VectorSubcoreMesh(core_axis_name="c",
                                    subcore_axis_name="s"),
        compiler_params=pltpu.CompilerParams(needs_layout_passes=False),
        scratch_types=[
            pltpu.VMEM((TABLE_LEN,), jnp.float32),
            pltpu.VMEM((CODES_PER_CHUNK,), jnp.int32),
            pltpu.VMEM((CODES_PER_CHUNK,), jnp.int32),
            pltpu.VMEM((CHUNK_ROWS, COLS), jnp.float32),
            pltpu.VMEM((CHUNK_ROWS, COLS), jnp.float32),
            pltpu.SemaphoreType.DMA,
            pltpu.SemaphoreType.DMA,
            pltpu.SemaphoreType.DMA,
            pltpu.SemaphoreType.DMA,
            pltpu.SemaphoreType.DMA,
        ],
    )(body)


_dequant_sc = [_make_dequant_half(0), _make_dequant_half(1)]


BM, BN = 512, 2048


def _addmm_body0(dq_ref, sc_ref, l_ref, r_ref, o_ref):
    lr = jnp.dot(l_ref[...], r_ref[...],
                 preferred_element_type=jnp.float32)
    o_ref[...] = dq_ref[...] * sc_ref[...] + lr


def _addmm_body1(prev_ref, dq_ref, sc_ref, l_ref, r_ref, o_ref):
    del prev_ref
    _addmm_body0(dq_ref, sc_ref, l_ref, r_ref, o_ref)


def _addmm_tc(half, dq, scales, L16, R16, prev=None):
    row_off = half * (HROWS // BM)
    data_specs = [
        pl.BlockSpec((BM, BN), lambda i, j: (i, j)),
        pl.BlockSpec((BM, 1), lambda i, j: (i + row_off, 0)),
        pl.BlockSpec((BM, RANK), lambda i, j: (i + row_off, 0)),
        pl.BlockSpec((RANK, BN), lambda i, j: (0, j)),
    ]
    kwargs = {}
    if half == 0:
        body = _addmm_body0
        in_specs = data_specs
        args = (dq, scales, L16, R16)
    else:
        body = _addmm_body1
        in_specs = [pl.BlockSpec(memory_space=pltpu.MemorySpace.HBM)] + data_specs
        args = (prev, dq, scales, L16, R16)
        kwargs["input_output_aliases"] = {0: 0}
    return pl.pallas_call(
        body,
        grid=(HROWS // BM, COLS // BN),
        in_specs=in_specs,
        out_specs=pl.BlockSpec((BM, BN), lambda i, j: (i + row_off, j)),
        out_shape=jax.ShapeDtypeStruct((ROWS, COLS), jnp.float32),
        compiler_params=pltpu.CompilerParams(
            dimension_semantics=("parallel", "parallel")),
        **kwargs,
    )(*args)


def kernel(codebooks, codes, scales, L, R):
    cb_flat = codebooks.reshape(-1)
    L16 = L.astype(jnp.bfloat16)
    R16 = R.astype(jnp.bfloat16)
    dq0 = _dequant_sc[0](cb_flat, codes)
    dq1 = _dequant_sc[1](cb_flat, codes)
    out = _addmm_tc(0, dq0, scales, L16, R16)
    out = _addmm_tc(1, dq1, scales, L16, R16, prev=out)
    return out


# speedup vs baseline: 1.0343x; 1.0343x over previous
import functools

import jax
import jax.numpy as jnp
from jax import lax
from jax.experimental import pallas as pl
from jax.experimental.pallas import tpu as pltpu
from jax.experimental.pallas import tpu_sc as plsc

ROWS, COLS = 4096, 4096
NUM_CODEBOOKS = 2
CODEBOOK_SIZE = 256
CENTROID_LEN = 8
RANK = 256

NC, NS, LANES = 2, 16, 16
NW = NC * NS
HROWS = ROWS // NUM_CODEBOOKS
N_CODES_HALF = HROWS * COLS // CENTROID_LEN
CODES_PER_W = N_CODES_HALF // NW

CODES_PER_CHUNK = 4096
CHUNK_ROWS = CODES_PER_CHUNK * CENTROID_LEN // COLS
NCHUNK = CODES_PER_W // CODES_PER_CHUNK
GROUPS_PER_CHUNK = CODES_PER_CHUNK // LANES
GROUPS_PER_ROW = COLS // (LANES * CENTROID_LEN)
ROWS_PER_W = HROWS // NW
TABLE_LEN = NUM_CODEBOOKS * CODEBOOK_SIZE * CENTROID_LEN


def _make_dequant_half(half):
    half_off = half * CODEBOOK_SIZE * CENTROID_LEN

    def body(cb_hbm, codes_hbm, dq_hbm,
             cb_v, codes_v0, codes_v1, out_v0, out_v1,
             sem_cb, sem_in0, sem_in1, sem_out0, sem_out1):
        wid = lax.axis_index("s") * NC + lax.axis_index("c")
        code_base = wid * CODES_PER_W
        row_base = wid * ROWS_PER_W

        pltpu.async_copy(cb_hbm, cb_v, sem_cb).wait()
        off8 = lax.iota(jnp.int32, LANES) * CENTROID_LEN

        sem_in = [sem_in0, sem_in1]
        sem_out = [sem_out0, sem_out1]
        codes_v = [codes_v0, codes_v1]
        out_v = [out_v0, out_v1]

        def start_in(ci):
            return pltpu.async_copy(
                codes_hbm.at[half,
                             pl.ds(code_base + ci * CODES_PER_CHUNK,
                                   CODES_PER_CHUNK)],
                codes_v[ci % 2], sem_in[ci % 2])

        def start_out(ci):
            return pltpu.async_copy(
                out_v[ci % 2],
                dq_hbm.at[pl.ds(row_base + ci * CHUNK_ROWS, CHUNK_ROWS)],
                sem_out[ci % 2])

        in_d = {0: start_in(0), 1: start_in(1)}
        out_d = {}
        for ci in range(NCHUNK):
            b = ci % 2
            in_d[ci].wait()
            if ci - 2 in out_d:
                out_d[ci - 2].wait()
            codes_b = codes_v[b]
            out_b = out_v[b]

            @plsc.parallel_loop(0, GROUPS_PER_CHUNK, unroll=4)
            def group(gi):
                codes16 = codes_b[pl.ds(gi * LANES, LANES)]
                eidx = codes16 * CENTROID_LEN + half_off
                row16 = jnp.full((LANES,), gi >> 5, jnp.int32)
                cbase = (gi & (GROUPS_PER_ROW - 1)) * (LANES * CENTROID_LEN)
                for k in range(CENTROID_LEN):
                    vals = plsc.load_gather(cb_v, [eidx + k])
                    plsc.store_scatter(
                        out_b, [row16, off8 + (cbase + k)], vals)

            out_d[ci] = start_out(ci)
            if ci + 2 < NCHUNK:
                in_d[ci + 2] = start_in(ci + 2)
        out_d[NCHUNK - 2].wait()
        out_d[NCHUNK - 1].wait()

    return functools.partial(
        pl.kernel,
        out_type=jax.ShapeDtypeStruct((HROWS, COLS), jnp.float32),
        mesh=plsc.VectorSubcoreMesh(core_axis_name="c",
                                    subcore_axis_name="s"),
        compiler_params=pltpu.CompilerParams(needs_layout_passes=False),
        scratch_types=[
            pltpu.VMEM((TABLE_LEN,), jnp.float32),
            pltpu.VMEM((CODES_PER_CHUNK,), jnp.int32),
            pltpu.VMEM((CODES_PER_CHUNK,), jnp.int32),
            pltpu.VMEM((CHUNK_ROWS, COLS), jnp.float32),
            pltpu.VMEM((CHUNK_ROWS, COLS), jnp.float32),
            pltpu.SemaphoreType.DMA,
            pltpu.SemaphoreType.DMA,
            pltpu.SemaphoreType.DMA,
            pltpu.SemaphoreType.DMA,
            pltpu.SemaphoreType.DMA,
        ],
    )(body)


_dequant_sc = [_make_dequant_half(0), _make_dequant_half(1)]


BM, BN = 512, 2048


def _addmm_body0(dq_ref, sc_ref, l_ref, r_ref, o_ref):
    lr = jnp.dot(l_ref[...], r_ref[...],
                 preferred_element_type=jnp.float32)
    o_ref[...] = dq_ref[...] * sc_ref[...] + lr


def _addmm_body1(prev_ref, dq_ref, sc_ref, l_ref, r_ref, o_ref):
    del prev_ref
    _addmm_body0(dq_ref, sc_ref, l_ref, r_ref, o_ref)


def _addmm_tc(half, dq, scales, L16, R16, prev=None):
    row_off = half * (HROWS // BM)
    data_specs = [
        pl.BlockSpec((BM, BN), lambda i, j: (i, j)),
        pl.BlockSpec((BM, 1), lambda i, j: (i + row_off, 0)),
        pl.BlockSpec((BM, RANK), lambda i, j: (i + row_off, 0)),
        pl.BlockSpec((RANK, BN), lambda i, j: (0, j)),
    ]
    kwargs = {}
    if half == 0:
        body = _addmm_body0
        in_specs = data_specs
        args = (dq, scales, L16, R16)
    else:
        body = _addmm_body1
        in_specs = [pl.BlockSpec(memory_space=pltpu.MemorySpace.HBM)] + data_specs
        args = (prev, dq, scales, L16, R16)
        kwargs["input_output_aliases"] = {0: 0}
    return pl.pallas_call(
        body,
        grid=(HROWS // BM, COLS // BN),
        in_specs=in_specs,
        out_specs=pl.BlockSpec((BM, BN), lambda i, j: (i + row_off, j)),
        out_shape=jax.ShapeDtypeStruct((ROWS, COLS), jnp.float32),
        compiler_params=pltpu.CompilerParams(
            dimension_semantics=("parallel", "parallel")),
        **kwargs,
    )(*args)


def kernel(codebooks, codes, scales, L, R):
    cb_flat = codebooks.reshape(-1)
    L16 = L.astype(jnp.bfloat16)
    R16 = R.astype(jnp.bfloat16)
    dq0 = _dequant_sc[0](cb_flat, codes)
    dq1 = _dequant_sc[1](cb_flat, codes)
    out = _addmm_tc(0, dq0, scales, L16, R16)
    out = _addmm_tc(1, dq1, scales, L16, R16, prev=out)
    return out
